# baseline (speedup 1.0000x reference)
import functools

import jax
import jax.numpy as jnp
from jax.experimental import pallas as pl
from jax.experimental.pallas import tpu as pltpu

NUM_USERS = 50000
NUM_ITEMS = 50000
LATENT_DIM = 32
N_LAYERS = 3
BATCH = 1024

USER_BLK = 64


def _rating_body(u_ref, i_ref, o_ref):
    u = u_ref[...]
    it = i_ref[...]
    acc = jax.lax.dot_general(u, it, (((1,), (1,)), ((), ())),
                              preferred_element_type=jnp.float32)
    o_ref[...] = jax.nn.sigmoid(acc)


@jax.jit
def _rating(users_emb, items_emb):
    n_items = items_emb.shape[0]
    grid = (BATCH // USER_BLK,)
    return pl.pallas_call(
        _rating_body,
        grid=grid,
        in_specs=[
            pl.BlockSpec((USER_BLK, LATENT_DIM), lambda j: (j, 0)),
            pl.BlockSpec((n_items, LATENT_DIM), lambda j: (0, 0)),
        ],
        out_specs=pl.BlockSpec((USER_BLK, n_items), lambda j: (j, 0)),
        out_shape=jax.ShapeDtypeStruct((BATCH, n_items), jnp.float32),
    )(users_emb, items_emb)


def kernel(users, edge_index, edge_values, user_emb, item_emb):
    n_nodes = NUM_USERS + NUM_ITEMS
    all_emb = jnp.concatenate([user_emb, item_emb], axis=0)
    src = edge_index[0]
    dst = edge_index[1]
    acc = all_emb
    emb = all_emb
    for _ in range(N_LAYERS):
        msgs = emb[src] * edge_values[:, None]
        emb = jax.ops.segment_sum(msgs, dst, num_segments=n_nodes)
        acc = acc + emb
    light_out = acc * (1.0 / (N_LAYERS + 1))
    users_emb = light_out[users]
    items_emb = light_out[NUM_USERS:]
    return _rating(users_emb, items_emb)
